# baseline (speedup 1.0000x reference)
import jax
import jax.numpy as jnp
from jax.experimental import pallas as pl
from jax.experimental.pallas import tpu as pltpu

_DIM = 128
_TOTAL = 32768
_NLEV = 15
_ILP = 8
_CMAX = 256


def _chunk_size(t):
    max_p = (_TOTAL >> (t + 1)) + 2
    c = 8
    while c < max_p and c < _CMAX:
        c *= 2
    return c


_MAXCH = _TOTAL // (2 * _CMAX) + 32


def _tree_kernel(limits_ref, argl_ref, argh_ref, w1t_ref, b1_ref, w2t_ref,
                 b2_ref, out_ref, al_ref, ah_ref, bl_ref, bh_ref,
                 cs_ref, ln_ref,
                 t00i_ref, t00o_ref, t01i_ref, t01o_ref,
                 t10i_ref, t10o_ref, t11i_ref, t11o_ref,
                 lsrc_ref, ldst_ref):
    nseg = limits_ref.shape[0] - 1

    def mlp(x2):
        h = jnp.dot(x2, w1t_ref[...], preferred_element_type=jnp.float32)
        h = jnp.maximum(h + b1_ref[...], 0.0)
        y = jnp.dot(h, w2t_ref[...], preferred_element_type=jnp.float32)
        return y + b2_ref[...]

    tabs = {(0, 0): (t00i_ref, t00o_ref), (0, 1): (t01i_ref, t01o_ref),
            (1, 0): (t10i_ref, t10o_ref), (1, 1): (t11i_ref, t11o_ref)}

    for t in range(_NLEV):
        C = _chunk_size(t)
        combos = [(0, 0), (0, 1), (1, 0), (1, 1)] if t == 0 else [(0, 0), (0, 1)]
        if t == 0:
            in_lo, in_hi = argl_ref, argh_ref
        elif t % 2 == 0:
            in_lo, in_hi = bl_ref, bh_ref
        else:
            in_lo, in_hi = al_ref, ah_ref
        dst_lo, dst_hi = (al_ref, ah_ref) if t % 2 == 0 else (bl_ref, bh_ref)

        def build_seg(s, carry, t=t, C=C, combos=combos):
            counts = dict(zip(combos, carry[:len(combos)]))
            li, dcum = carry[len(combos):]
            if t == 0:
                src = limits_ref[s]
                length = limits_ref[s + 1] - src
            else:
                src = cs_ref[s]
                length = ln_ref[s]
            p = length // 2
            odd = length - 2 * p
            nch = (p + C - 1) // C
            rp = src % 2 if t == 0 else 0
            new_counts = {}
            for (crp, cwp), ci in counts.items():
                is_c = (odd == cwp) if (t != 0 or len(combos) == 2) else \
                    jnp.logical_and(rp == crp, odd == cwp)

                @pl.when(jnp.logical_and(is_c, nch > 0))
                def _(ci=ci, crp=crp, cwp=cwp):
                    ti, to = tabs[(crp, cwp)]

                    def put(j, _):
                        ti[ci + j] = (src >> 1) + C * j
                        to[ci + j] = (dcum >> 1) + (C // 2) * j
                        return 0
                    jax.lax.fori_loop(0, nch, put, 0)

                new_counts[(crp, cwp)] = ci + jnp.where(is_c, nch, 0)

            @pl.when(odd == 1)
            def _():
                lsrc_ref[li] = src + 2 * p
                ldst_ref[li] = dcum >> 1

            li = li + odd
            lnew = p + odd
            cs_ref[s] = dcum
            ln_ref[s] = lnew
            rsize = lnew + C
            rsize = rsize + (rsize % 2)
            return tuple(new_counts[c] for c in combos) + (li, dcum + rsize)

        z = jnp.int32(0)
        res = jax.lax.fori_loop(0, nseg, build_seg,
                                (z,) * len(combos) + (z, z))
        combo_counts = res[:len(combos)]
        nleft = res[len(combos)]

        for (crp, cwp), nch_all in zip(combos, combo_counts):
            ti, to = tabs[(crp, cwp)]

            def pad_dup(k, _, ti=ti, to=to, nch_all=nch_all):
                @pl.when(k >= nch_all)
                def _():
                    ti[k] = ti[nch_all - 1]
                    to[k] = to[nch_all - 1]
                return 0

            npad = (nch_all + _ILP - 1) // _ILP * _ILP

            @pl.when(nch_all > 0)
            def _(nch_all=nch_all, npad=npad, pad_dup=pad_dup):
                jax.lax.fori_loop(nch_all, npad, pad_dup, 0)

            def chunk_group(c, _, C=C, in_lo=in_lo, in_hi=in_hi,
                            dst_lo=dst_lo, dst_hi=dst_hi,
                            ti=ti, to=to, crp=crp, cwp=cwp):
                for u in range(_ILP):
                    k = _ILP * c + u
                    ir = ti[k]
                    orow = to[k]
                    if crp == 0:
                        xa = in_lo[pl.ds(ir, C), :]
                        xb = in_hi[pl.ds(ir, C), :]
                    else:
                        xa = in_hi[pl.ds(ir, C), :]
                        xb = in_lo[pl.ds(ir + 1, C), :]
                    x2 = jnp.concatenate([xa, xb], axis=1)
                    z = mlp(x2).reshape(C // 2, 2 * _DIM)
                    if cwp == 0:
                        dst_lo[pl.ds(orow, C // 2), :] = z[:, 0:128]
                        dst_hi[pl.ds(orow, C // 2), :] = z[:, 128:256]
                    else:
                        dst_hi[pl.ds(orow, C // 2), :] = z[:, 0:128]
                        dst_lo[pl.ds(orow + 1, C // 2), :] = z[:, 128:256]
                return 0

            jax.lax.fori_loop(0, npad // _ILP, chunk_group, 0)

        def left_copy(k, _, in_lo=in_lo, in_hi=in_hi, dst_lo=dst_lo):
            e = lsrc_ref[k]
            r = e >> 1
            d = ldst_ref[k]

            @pl.when(e % 2 == 0)
            def _():
                dst_lo[pl.ds(d, 1), :] = in_lo[pl.ds(r, 1), :]

            @pl.when(e % 2 == 1)
            def _():
                dst_lo[pl.ds(d, 1), :] = in_hi[pl.ds(r, 1), :]
            return 0

        jax.lax.fori_loop(0, nleft, left_copy, 0)

    def out_seg(s, _):
        out_ref[pl.ds(s, 1), :] = al_ref[pl.ds(cs_ref[s] >> 1, 1), :]
        return 0

    jax.lax.fori_loop(0, nseg, out_seg, 0)


def kernel(args, limits, W1, b1, W2, b2):
    total, dim = args.shape
    nseg = limits.shape[0] - 1
    c0 = _chunk_size(0)
    args_p = jnp.concatenate(
        [args, jnp.zeros((2 * c0, dim), args.dtype)], axis=0)
    args_lo = args_p[0::2]
    args_hi = args_p[1::2]
    elem_cap = (total + nseg) // 2 + (nseg + 3) * c0
    rows2 = elem_cap // 2 + c0
    rows2 = (rows2 + 7) // 8 * 8

    out = pl.pallas_call(
        _tree_kernel,
        out_shape=jax.ShapeDtypeStruct((nseg, dim), jnp.float32),
        in_specs=[
            pl.BlockSpec(memory_space=pltpu.SMEM),
            pl.BlockSpec(memory_space=pltpu.VMEM),
            pl.BlockSpec(memory_space=pltpu.VMEM),
            pl.BlockSpec(memory_space=pltpu.VMEM),
            pl.BlockSpec(memory_space=pltpu.VMEM),
            pl.BlockSpec(memory_space=pltpu.VMEM),
            pl.BlockSpec(memory_space=pltpu.VMEM),
        ],
        out_specs=pl.BlockSpec(memory_space=pltpu.VMEM),
        scratch_shapes=[
            pltpu.VMEM((rows2, dim), jnp.float32),
            pltpu.VMEM((rows2, dim), jnp.float32),
            pltpu.VMEM((rows2, dim), jnp.float32),
            pltpu.VMEM((rows2, dim), jnp.float32),
            pltpu.SMEM((nseg,), jnp.int32),
            pltpu.SMEM((nseg,), jnp.int32),
            pltpu.SMEM((_MAXCH,), jnp.int32),
            pltpu.SMEM((_MAXCH,), jnp.int32),
            pltpu.SMEM((_MAXCH,), jnp.int32),
            pltpu.SMEM((_MAXCH,), jnp.int32),
            pltpu.SMEM((_MAXCH,), jnp.int32),
            pltpu.SMEM((_MAXCH,), jnp.int32),
            pltpu.SMEM((_MAXCH,), jnp.int32),
            pltpu.SMEM((_MAXCH,), jnp.int32),
            pltpu.SMEM((nseg,), jnp.int32),
            pltpu.SMEM((nseg,), jnp.int32),
        ],
    )(
        limits.astype(jnp.int32),
        args_lo,
        args_hi,
        W1.T,
        b1.reshape(1, -1),
        W2.T,
        b2.reshape(1, -1),
    )
    return out


# speedup vs baseline: 2.4083x; 2.4083x over previous
import jax
import jax.numpy as jnp
from jax.experimental import pallas as pl
from jax.experimental.pallas import tpu as pltpu

_DIM = 128
_TOTAL = 32768
_NLEV = 15
_ILP = 16


_CMAX = 256


def _chunk_size(t):
    max_p = (_TOTAL >> (t + 1)) + 2
    c = 8
    while c < max_p and c < _CMAX:
        c *= 2
    return c


_MAXCH = _TOTAL // (2 * _CMAX) + 32


def _tree_kernel(limits_ref, args_ref, w1t_ref, b1_ref, w2t_ref, b2_ref,
                 out_ref, bufa_ref, bufb_ref,
                 cs_ref, ln_ref, tin_ref, tout_ref, lsrc_ref, ldst_ref):
    nseg = limits_ref.shape[0] - 1

    def mlp(x2):
        h = jnp.dot(x2, w1t_ref[...], preferred_element_type=jnp.float32)
        h = jnp.maximum(h + b1_ref[...], 0.0)
        y = jnp.dot(h, w2t_ref[...], preferred_element_type=jnp.float32)
        return y + b2_ref[...]

    for t in range(_NLEV):
        C = _chunk_size(t)
        in_ref = args_ref if t == 0 else (bufb_ref if t % 2 == 0 else bufa_ref)
        dst_buf = bufa_ref if t % 2 == 0 else bufb_ref

        def build_seg(s, carry, t=t, C=C):
            ci, li, dcum = carry
            if t == 0:
                src = limits_ref[s]
                length = limits_ref[s + 1] - src
            else:
                src = cs_ref[s]
                length = ln_ref[s]
            p = length // 2
            odd = length - 2 * p
            nch = (p + C - 1) // C

            def put(j, ci):
                tin_ref[ci] = src + 2 * C * j
                tout_ref[ci] = dcum + odd + C * j
                return ci + 1

            ci = jax.lax.fori_loop(0, nch, put, ci)

            @pl.when(odd == 1)
            def _():
                lsrc_ref[li] = src + 2 * p
                ldst_ref[li] = dcum

            li = li + odd
            cs_ref[s] = dcum
            ln_ref[s] = p + odd
            return ci, li, dcum + p + odd + C

        nch_all, nleft, _ = jax.lax.fori_loop(
            0, nseg, build_seg, (jnp.int32(0), jnp.int32(0), jnp.int32(0)))

        def pad_dup(k, _):
            @pl.when(k >= nch_all)
            def _():
                tin_ref[k] = tin_ref[nch_all - 1]
                tout_ref[k] = tout_ref[nch_all - 1]
            return 0

        npad = (nch_all + _ILP - 1) // _ILP * _ILP

        @pl.when(nch_all > 0)
        def _():
            jax.lax.fori_loop(nch_all, npad, pad_dup, 0)

        def chunk_group(c, _, C=C, in_ref=in_ref, dst_buf=dst_buf):
            for u in range(_ILP):
                k = _ILP * c + u
                ib = tin_ref[k]
                ob = tout_ref[k]
                x = in_ref[pl.ds(ib, 2 * C), :]
                dst_buf[pl.ds(ob, C), :] = mlp(x.reshape(C, 2 * _DIM))
            return 0

        jax.lax.fori_loop(0, npad // _ILP, chunk_group, 0)

        def left_copy(k, _, in_ref=in_ref, dst_buf=dst_buf):
            dst_buf[pl.ds(ldst_ref[k], 1), :] = in_ref[pl.ds(lsrc_ref[k], 1), :]
            return 0

        jax.lax.fori_loop(0, nleft, left_copy, 0)

    def out_seg(s, _):
        out_ref[pl.ds(s, 1), :] = bufa_ref[pl.ds(cs_ref[s], 1), :]
        return 0

    jax.lax.fori_loop(0, nseg, out_seg, 0)


def kernel(args, limits, W1, b1, W2, b2):
    total, dim = args.shape
    nseg = limits.shape[0] - 1
    c0 = _chunk_size(0)
    args_p = jnp.concatenate(
        [args, jnp.zeros((2 * c0, dim), args.dtype)], axis=0)
    buf_rows = (total + nseg) // 2 + (nseg + 3) * c0
    buf_rows = (buf_rows + 7) // 8 * 8

    out = pl.pallas_call(
        _tree_kernel,
        out_shape=jax.ShapeDtypeStruct((nseg, dim), jnp.float32),
        in_specs=[
            pl.BlockSpec(memory_space=pltpu.SMEM),
            pl.BlockSpec(memory_space=pltpu.VMEM),
            pl.BlockSpec(memory_space=pltpu.VMEM),
            pl.BlockSpec(memory_space=pltpu.VMEM),
            pl.BlockSpec(memory_space=pltpu.VMEM),
            pl.BlockSpec(memory_space=pltpu.VMEM),
        ],
        out_specs=pl.BlockSpec(memory_space=pltpu.VMEM),
        scratch_shapes=[
            pltpu.VMEM((buf_rows, dim), jnp.float32),
            pltpu.VMEM((buf_rows, dim), jnp.float32),
            pltpu.SMEM((nseg,), jnp.int32),
            pltpu.SMEM((nseg,), jnp.int32),
            pltpu.SMEM((_MAXCH,), jnp.int32),
            pltpu.SMEM((_MAXCH,), jnp.int32),
            pltpu.SMEM((nseg,), jnp.int32),
            pltpu.SMEM((nseg,), jnp.int32),
        ],
    )(
        limits.astype(jnp.int32),
        args_p,
        W1.T,
        b1.reshape(1, -1),
        W2.T,
        b2.reshape(1, -1),
    )
    return out


# speedup vs baseline: 2.9510x; 1.2253x over previous
import jax
import jax.numpy as jnp
from jax.experimental import pallas as pl
from jax.experimental.pallas import tpu as pltpu

_DIM = 128
_TOTAL = 32768
_NLEV = 15
_ILP = 16


_CMAX = 256


def _chunk_size(t):
    max_p = (_TOTAL >> (t + 1)) + 2
    c = 8
    while c < max_p and c < _CMAX:
        c *= 2
    return c


_MAXCH = _TOTAL // (2 * _CMAX) + 32


def _tree_kernel(limits_ref, args_ref, w1t_ref, b1_ref, w2t_ref, b2_ref,
                 out_ref, bufa_ref, bufb_ref,
                 cs_ref, ln_ref, tin_ref, tout_ref, lsrc_ref, ldst_ref):
    nseg = limits_ref.shape[0] - 1

    def mlp(x2):
        h = jnp.dot(x2, w1t_ref[...], preferred_element_type=jnp.float32)
        h = jnp.maximum(h + b1_ref[...], 0.0)
        y = jnp.dot(h, w2t_ref[...], preferred_element_type=jnp.float32)
        return y + b2_ref[...]

    for t in range(_NLEV):
        C = _chunk_size(t)
        in_ref = args_ref if t == 0 else (bufb_ref if t % 2 == 0 else bufa_ref)
        dst_buf = bufa_ref if t % 2 == 0 else bufb_ref

        def build_seg(s, carry, t=t, C=C):
            ci, li, dcum = carry
            if t == 0:
                src = limits_ref[s]
                length = limits_ref[s + 1] - src
            else:
                src = cs_ref[s]
                length = ln_ref[s]
            p = length // 2
            odd = length - 2 * p
            nch = (p + C - 1) // C

            def put(j, ci):
                if t == 0:
                    offp = jnp.maximum(jnp.minimum(C * j, p - C),
                                       -(src >> 1))
                else:
                    offp = C * j
                tin_ref[ci] = src + 2 * offp
                tout_ref[ci] = dcum + odd + offp
                return ci + 1

            ci = jax.lax.fori_loop(0, nch, put, ci)

            @pl.when(odd == 1)
            def _():
                lsrc_ref[li] = src + 2 * p
                ldst_ref[li] = dcum

            li = li + odd
            cs_ref[s] = dcum
            ln_ref[s] = p + odd
            return ci, li, dcum + p + odd + C

        nch_all, nleft, _ = jax.lax.fori_loop(
            0, nseg, build_seg, (jnp.int32(0), jnp.int32(0), jnp.int32(0)))

        def pad_dup(k, _):
            @pl.when(k >= nch_all)
            def _():
                tin_ref[k] = tin_ref[nch_all - 1]
                tout_ref[k] = tout_ref[nch_all - 1]
            return 0

        npad = (nch_all + _ILP - 1) // _ILP * _ILP

        @pl.when(nch_all > 0)
        def _():
            jax.lax.fori_loop(nch_all, npad, pad_dup, 0)

        def chunk_group(c, _, C=C, in_ref=in_ref, dst_buf=dst_buf):
            for u in range(_ILP):
                k = _ILP * c + u
                ib = tin_ref[k]
                ob = tout_ref[k]
                x = in_ref[pl.ds(ib, 2 * C), :]
                dst_buf[pl.ds(ob, C), :] = mlp(x.reshape(C, 2 * _DIM))
            return 0

        jax.lax.fori_loop(0, npad // _ILP, chunk_group, 0)

        def left_copy(k, _, in_ref=in_ref, dst_buf=dst_buf):
            dst_buf[pl.ds(ldst_ref[k], 1), :] = in_ref[pl.ds(lsrc_ref[k], 1), :]
            return 0

        jax.lax.fori_loop(0, nleft, left_copy, 0)

    def out_seg(s, _):
        out_ref[pl.ds(s, 1), :] = bufa_ref[pl.ds(cs_ref[s], 1), :]
        return 0

    jax.lax.fori_loop(0, nseg, out_seg, 0)


def kernel(args, limits, W1, b1, W2, b2):
    total, dim = args.shape
    nseg = limits.shape[0] - 1
    c0 = _chunk_size(0)
    buf_rows = (total + nseg) // 2 + (nseg + 3) * c0
    buf_rows = (buf_rows + 7) // 8 * 8

    out = pl.pallas_call(
        _tree_kernel,
        out_shape=jax.ShapeDtypeStruct((nseg, dim), jnp.float32),
        in_specs=[
            pl.BlockSpec(memory_space=pltpu.SMEM),
            pl.BlockSpec(memory_space=pltpu.VMEM),
            pl.BlockSpec(memory_space=pltpu.VMEM),
            pl.BlockSpec(memory_space=pltpu.VMEM),
            pl.BlockSpec(memory_space=pltpu.VMEM),
            pl.BlockSpec(memory_space=pltpu.VMEM),
        ],
        out_specs=pl.BlockSpec(memory_space=pltpu.VMEM),
        scratch_shapes=[
            pltpu.VMEM((buf_rows, dim), jnp.float32),
            pltpu.VMEM((buf_rows, dim), jnp.float32),
            pltpu.SMEM((nseg,), jnp.int32),
            pltpu.SMEM((nseg,), jnp.int32),
            pltpu.SMEM((_MAXCH,), jnp.int32),
            pltpu.SMEM((_MAXCH,), jnp.int32),
            pltpu.SMEM((nseg,), jnp.int32),
            pltpu.SMEM((nseg,), jnp.int32),
        ],
    )(
        limits.astype(jnp.int32),
        args,
        W1.T,
        b1.reshape(1, -1),
        W2.T,
        b2.reshape(1, -1),
    )
    return out
